# baseline (speedup 1.0000x reference)
import functools

import numpy as np

import jax
import jax.numpy as jnp
from jax import lax
from jax.experimental import pallas as pl
from jax.experimental.pallas import tpu as pltpu
import jax.experimental.pallas.tpu_sc as plsc

BN = 1000
NC, NS, L = 2, 16, 16
NW = NC * NS
CH = 80


def _mlp_kernel(x_ref, w1_ref, b1_ref, w2_ref, b2_ref,
                gf_ref, gw1_ref, gb1_ref, gw2_ref, gb2_ref,
                h2_ref, g_ref):
    x = x_ref[...]
    h = jnp.maximum(
        jnp.dot(x, w1_ref[...], preferred_element_type=jnp.float32)
        + b1_ref[...], 0.0)
    h2_ref[...] = (jnp.dot(h, w2_ref[...], preferred_element_type=jnp.float32)
                   + b2_ref[...]).astype(jnp.bfloat16)

    @pl.when(pl.program_id(0) == 0)
    def _():
        gh = jnp.maximum(
            jnp.dot(gf_ref[...], gw1_ref[...],
                    preferred_element_type=jnp.float32) + gb1_ref[...], 0.0)
        g_ref[...] = (jnp.dot(gh, gw2_ref[...],
                              preferred_element_type=jnp.float32)
                      + gb2_ref[...])


def _combine_kernel(x_ref, msg_ref, den_ref, g_ref, unperm_ref,
                    w1_ref, b1_ref, w2_ref, b2_ref, out_ref):
    x = x_ref[...]
    h = jnp.maximum(
        jnp.dot(x, w1_ref[...], preferred_element_type=jnp.float32)
        + b1_ref[...], 0.0)
    h1 = (jnp.dot(h, w2_ref[...], preferred_element_type=jnp.float32)
          + b2_ref[...])
    del unperm_ref
    msg = msg_ref[0] + msg_ref[1]
    den = jnp.sum(den_ref[...], axis=1)[:, None] + 1e-07
    inter = h1 + msg / den + g_ref[...]
    mean = jnp.mean(inter, axis=1, keepdims=True)
    var = jnp.mean((inter - mean) ** 2, axis=1, keepdims=True)
    normed = (inter - mean) * lax.rsqrt(var + 1e-05)
    out_ref[...] = x + jnp.maximum(normed, 0.0)


def _full_spec(shape):
    return pl.BlockSpec(shape, lambda i: (0,) * len(shape))


def _sc_edge_body(Np, N, d, NCH,
                  h2_hbm, epk_hbm, msg_hbm, den_hbm,
                  accum_sh, eba, ebb, sidxa, sidxb, siga, sigb,
                  rowsa, rowsb, sbufa, sbufb, denv, sem_e, sem_g, sem_s):
    c = lax.axis_index("c")
    s = lax.axis_index("s")
    wid = c * NS + s
    rows_per_s = Np // NS
    nwb = rows_per_s // CH

    z16 = jnp.zeros((L,), jnp.float32)

    def zero_den(i, _):
        denv[pl.ds(pl.multiple_of(i * L, L), L)] = z16
        return 0
    lax.fori_loop(0, N // L, zero_den, 0)

    def zero_rows(i, _):
        for j in range(d // L):
            sbufa[i, pl.ds(j * L, L)] = z16
        return 0
    lax.fori_loop(0, CH, zero_rows, 0)
    for k in range(nwb):
        pltpu.sync_copy(sbufa, accum_sh.at[pl.ds(s * rows_per_s + k * CH, CH)])
    plsc.subcore_barrier()

    def wait_gather(sem):
        pltpu.make_async_copy(h2_hbm.at[pl.ds(0, CH)], rowsa, sem).wait()

    def wait_scatter(sem):
        pltpu.make_async_copy(msg_hbm.at[0, pl.ds(0, CH)], sbufa, sem).wait()

    def wait_eb(sem):
        pltpu.make_async_copy(epk_hbm.at[0, 0], eba, sem).wait()

    def issue_e(ci, eb):
        pltpu.async_copy(epk_hbm.at[wid, ci], eb, sem_e)

    def issue_g(ci, eb, rows):
        pltpu.async_copy(h2_hbm.at[eb.at[1]], rows, sem_g)

    def consume_eb(eb, sidx, sig):
        for k in range(CH // L):
            sl = pl.ds(k * L, L)
            srcv = eb[0, sl]
            sidx[sl] = srcv
            efv = lax.bitcast_convert_type(eb[2, sl], jnp.float32)
            sg = 1.0 / (1.0 + jnp.exp(-efv))
            sig[sl] = sg
            plsc.addupdate_scatter(denv, [srcv], sg)

    himask = jnp.int32(-65536)

    def scale(rows, sbuf, sig):
        def scale_group(gi, _):
            sg16 = sig[pl.ds(pl.multiple_of(gi * L, L), L)]
            rbase = gi * L
            for rr in range(L):
                sv = sg16[rr]
                for grp in range(d // (2 * L)):
                    w = plsc.bitcast(
                        rows[rbase + rr, pl.ds(grp * 2 * L, 2 * L)],
                        jnp.int32)
                    lo = plsc.bitcast(w << 16, jnp.float32)
                    hi = plsc.bitcast(w & himask, jnp.float32)
                    sbuf[rbase + rr, pl.ds(grp * 2 * L, L)] = lo * sv
                    sbuf[rbase + rr, pl.ds(grp * 2 * L + L, L)] = hi * sv
            return 0
        lax.fori_loop(0, CH // L, scale_group, 0)

    def step(ci, ebp, sidxp, sigp, rowsp, sbufp, ebq, rowsq):
        wait_gather(sem_g)
        consume_eb(ebp, sidxp, sigp)

        @pl.when(ci + 2 < NCH)
        def _():
            issue_e(ci + 2, ebp)

        @pl.when(ci > 0)
        def _():
            wait_scatter(sem_s)

        @pl.when(ci + 1 < NCH)
        def _():
            wait_eb(sem_e)
            issue_g(ci + 1, ebq, rowsq)

        scale(rowsp, sbufp, sigp)
        pltpu.async_copy(sbufp, accum_sh.at[sidxp], sem_s, add=True)

    issue_e(0, eba)
    wait_eb(sem_e)
    issue_g(0, eba, rowsa)
    issue_e(1, ebb)

    def pair_body(t, _):
        c1 = 2 * t
        step(c1, eba, sidxa, siga, rowsa, sbufa, ebb, rowsb)
        step(c1 + 1, ebb, sidxb, sigb, rowsb, sbufb, eba, rowsa)
        return 0

    lax.fori_loop(0, NCH // 2, pair_body, 0)
    wait_scatter(sem_s)
    plsc.subcore_barrier()

    for k in range(nwb):
        start = s * rows_per_s + k * CH
        buf = sbufa if k % 2 == 0 else sbufb
        pltpu.sync_copy(accum_sh.at[pl.ds(start, CH)], buf)
        pltpu.sync_copy(buf, msg_hbm.at[c, pl.ds(start, CH)])
    pltpu.sync_copy(denv, den_hbm.at[wid])


def _sc_edge(h2, epk, N):
    d = h2.shape[1]
    NCH = epk.shape[1]
    Np = -(-N // (NS * 128)) * (NS * 128)
    mesh = plsc.VectorSubcoreMesh(core_axis_name="c", subcore_axis_name="s")
    f = pl.kernel(
        functools.partial(_sc_edge_body, Np, N, d, NCH),
        out_type=(jax.ShapeDtypeStruct((NC, Np, d), jnp.float32),
                  jax.ShapeDtypeStruct((NW, N), jnp.float32)),
        mesh=mesh,
        scratch_types=[
            pltpu.VMEM_SHARED((Np, d), jnp.float32),
            pltpu.VMEM((3, CH), jnp.int32),
            pltpu.VMEM((3, CH), jnp.int32),
            pltpu.VMEM((CH,), jnp.int32),
            pltpu.VMEM((CH,), jnp.int32),
            pltpu.VMEM((CH,), jnp.float32),
            pltpu.VMEM((CH,), jnp.float32),
            pltpu.VMEM((CH, d), jnp.bfloat16),
            pltpu.VMEM((CH, d), jnp.bfloat16),
            pltpu.VMEM((CH, d), jnp.float32),
            pltpu.VMEM((CH, d), jnp.float32),
            pltpu.VMEM((N,), jnp.float32),
            pltpu.SemaphoreType.DMA,
            pltpu.SemaphoreType.DMA,
            pltpu.SemaphoreType.DMA,
        ],
        compiler_params=pltpu.CompilerParams(needs_layout_passes=False,
                                             use_tc_tiling_on_sc=False),
    )
    return f(h2, epk)


def kernel(node_features, edge_index, edge_features, global_features,
           W1a, b1a, W2a, b2a, W1b, b1b, W2b, b2b, W1c, b1c, W2c, b2c):
    x = node_features[0]
    N, d = x.shape
    hdim = W1a.shape[0]
    src = edge_index[0, 0]
    dst = edge_index[0, 1]
    E = src.shape[0]
    NCH = 2 * E // (NW * CH)

    grid = N // BN
    row_spec = pl.BlockSpec((BN, d), lambda i: (i, 0))

    perm = np.empty((d,), np.int32)
    for gidx in range(d // 32):
        for k in range(16):
            perm[32 * gidx + 2 * k] = 32 * gidx + k
            perm[32 * gidx + 2 * k + 1] = 32 * gidx + 16 + k
    unperm = jnp.asarray(np.eye(d, dtype=np.float32)[perm])

    h2, g = pl.pallas_call(
        _mlp_kernel,
        grid=(grid,),
        in_specs=[
            row_spec,
            _full_spec((d, hdim)), _full_spec((1, hdim)),
            _full_spec((hdim, d)), _full_spec((1, d)),
            _full_spec((1, d)),
            _full_spec((d, hdim)), _full_spec((1, hdim)),
            _full_spec((hdim, d)), _full_spec((1, d)),
        ],
        out_specs=[row_spec, _full_spec((1, d))],
        out_shape=[jax.ShapeDtypeStruct((N, d), jnp.bfloat16),
                   jax.ShapeDtypeStruct((1, d), jnp.float32)],
    )(x, W1b.T, b1b[None], W2b.T[:, perm], b2b[None, perm],
      global_features[0], W1c.T, b1c[None], W2c.T, b2c[None])

    ebits = lax.bitcast_convert_type(edge_features[0], jnp.int32)
    epk = jnp.stack([
        jnp.concatenate([src, dst]).reshape(NW, NCH, CH),
        jnp.concatenate([dst, src]).reshape(NW, NCH, CH),
        jnp.concatenate([ebits, ebits]).reshape(NW, NCH, CH),
    ], axis=2)

    msg2, den32 = _sc_edge(h2, epk, N)

    out = pl.pallas_call(
        _combine_kernel,
        grid=(grid,),
        in_specs=[
            row_spec,
            pl.BlockSpec((NC, BN, d), lambda i: (0, i, 0)),
            pl.BlockSpec((BN, NW), lambda i: (i, 0)),
            _full_spec((1, d)), _full_spec((d, d)),
            _full_spec((d, hdim)), _full_spec((1, hdim)),
            _full_spec((hdim, d)), _full_spec((1, d)),
        ],
        out_specs=row_spec,
        out_shape=jax.ShapeDtypeStruct((N, d), jnp.float32),
    )(x, msg2, den32.T, g, unperm,
      W1a.T, b1a[None], W2a.T, b2a[None])

    return out[None]


# speedup vs baseline: 1.6146x; 1.6146x over previous
import functools

import jax
import jax.numpy as jnp
from jax import lax
from jax.experimental import pallas as pl
from jax.experimental.pallas import tpu as pltpu
import jax.experimental.pallas.tpu_sc as plsc

BN = 1000
NC, NS, L = 2, 16, 16
NW = NC * NS
CH = 80


def _mlp_kernel(x_ref, w1_ref, b1_ref, w2_ref, b2_ref,
                gf_ref, gw1_ref, gb1_ref, gw2_ref, gb2_ref,
                h2_ref, g_ref):
    x = x_ref[...]
    h = jnp.maximum(
        jnp.dot(x, w1_ref[...], preferred_element_type=jnp.float32)
        + b1_ref[...], 0.0)
    h2_ref[...] = (jnp.dot(h, w2_ref[...], preferred_element_type=jnp.float32)
                   + b2_ref[...])

    @pl.when(pl.program_id(0) == 0)
    def _():
        gh = jnp.maximum(
            jnp.dot(gf_ref[...], gw1_ref[...],
                    preferred_element_type=jnp.float32) + gb1_ref[...], 0.0)
        g_ref[...] = (jnp.dot(gh, gw2_ref[...],
                              preferred_element_type=jnp.float32)
                      + gb2_ref[...])


def _combine_kernel(x_ref, msg_ref, den_ref, g_ref,
                    w1_ref, b1_ref, w2_ref, b2_ref, out_ref):
    x = x_ref[...]
    h = jnp.maximum(
        jnp.dot(x, w1_ref[...], preferred_element_type=jnp.float32)
        + b1_ref[...], 0.0)
    h1 = (jnp.dot(h, w2_ref[...], preferred_element_type=jnp.float32)
          + b2_ref[...])
    msg = msg_ref[0] + msg_ref[1]
    den = jnp.sum(den_ref[...], axis=1)[:, None] + 1e-07
    inter = h1 + msg / den + g_ref[...]
    mean = jnp.mean(inter, axis=1, keepdims=True)
    var = jnp.mean((inter - mean) ** 2, axis=1, keepdims=True)
    normed = (inter - mean) * lax.rsqrt(var + 1e-05)
    out_ref[...] = x + jnp.maximum(normed, 0.0)


def _full_spec(shape):
    return pl.BlockSpec(shape, lambda i: (0,) * len(shape))


def _sc_edge_body(Np, N, d, NCH,
                  h2_hbm, epk_hbm, msg_hbm, den_hbm,
                  accum_sh, eba, ebb, sidxa, sidxb, siga, sigb,
                  rowsa, rowsb, denv, sem_e, sem_g, sem_s):
    c = lax.axis_index("c")
    s = lax.axis_index("s")
    wid = c * NS + s
    rows_per_s = Np // NS
    nwb = rows_per_s // CH

    z16 = jnp.zeros((L,), jnp.float32)

    def zero_den(i, _):
        denv[pl.ds(pl.multiple_of(i * L, L), L)] = z16
        return 0
    lax.fori_loop(0, N // L, zero_den, 0)

    def zero_rows(i, _):
        for j in range(d // L):
            rowsa[i, pl.ds(j * L, L)] = z16
        return 0
    lax.fori_loop(0, CH, zero_rows, 0)
    for k in range(nwb):
        pltpu.sync_copy(rowsa, accum_sh.at[pl.ds(s * rows_per_s + k * CH, CH)])
    plsc.subcore_barrier()

    def wait_gather(sem):
        pltpu.make_async_copy(h2_hbm.at[pl.ds(0, CH)], rowsa, sem).wait()

    def wait_scatter(sem):
        pltpu.make_async_copy(msg_hbm.at[0, pl.ds(0, CH)], rowsa, sem).wait()

    def wait_eb(sem):
        pltpu.make_async_copy(epk_hbm.at[0, 0], eba, sem).wait()

    def issue_e(ci, eb):
        pltpu.async_copy(epk_hbm.at[wid, ci], eb, sem_e)

    def issue_g(ci, eb, rows):
        pltpu.async_copy(h2_hbm.at[eb.at[1]], rows, sem_g)

    def consume_eb(eb, sidx, sig):
        for k in range(CH // L):
            sl = pl.ds(k * L, L)
            srcv = eb[0, sl]
            sidx[sl] = srcv
            efv = lax.bitcast_convert_type(eb[2, sl], jnp.float32)
            sg = 1.0 / (1.0 + jnp.exp(-efv))
            sig[sl] = sg
            plsc.addupdate_scatter(denv, [srcv], sg)

    def scale(rows, sig):
        def scale_group(gi, _):
            sg16 = sig[pl.ds(pl.multiple_of(gi * L, L), L)]
            rbase = gi * L
            for rr in range(L):
                sv = sg16[rr]
                for j in range(d // L):
                    rows[rbase + rr, pl.ds(j * L, L)] = (
                        rows[rbase + rr, pl.ds(j * L, L)] * sv)
            return 0
        lax.fori_loop(0, CH // L, scale_group, 0)

    def step(ci, ebp, sidxp, sigp, rowsp, ebq, rowsq):
        wait_gather(sem_g)
        consume_eb(ebp, sidxp, sigp)

        @pl.when(ci + 2 < NCH)
        def _():
            issue_e(ci + 2, ebp)

        @pl.when(ci > 0)
        def _():
            wait_scatter(sem_s)

        @pl.when(ci + 1 < NCH)
        def _():
            wait_eb(sem_e)
            issue_g(ci + 1, ebq, rowsq)

        scale(rowsp, sigp)
        pltpu.async_copy(rowsp, accum_sh.at[sidxp], sem_s, add=True)

    issue_e(0, eba)
    wait_eb(sem_e)
    issue_g(0, eba, rowsa)
    issue_e(1, ebb)

    def pair_body(t, _):
        c1 = 2 * t
        step(c1, eba, sidxa, siga, rowsa, ebb, rowsb)
        step(c1 + 1, ebb, sidxb, sigb, rowsb, eba, rowsa)
        return 0

    lax.fori_loop(0, NCH // 2, pair_body, 0)
    wait_scatter(sem_s)
    plsc.subcore_barrier()

    for k in range(nwb):
        start = s * rows_per_s + k * CH
        buf = rowsa if k % 2 == 0 else rowsb
        pltpu.sync_copy(accum_sh.at[pl.ds(start, CH)], buf)
        pltpu.sync_copy(buf, msg_hbm.at[c, pl.ds(start, CH)])
    pltpu.sync_copy(denv, den_hbm.at[wid])


def _sc_edge(h2, epk, N):
    d = h2.shape[1]
    NCH = epk.shape[1]
    Np = -(-N // (NS * 128)) * (NS * 128)
    mesh = plsc.VectorSubcoreMesh(core_axis_name="c", subcore_axis_name="s")
    f = pl.kernel(
        functools.partial(_sc_edge_body, Np, N, d, NCH),
        out_type=(jax.ShapeDtypeStruct((NC, Np, d), jnp.float32),
                  jax.ShapeDtypeStruct((NW, N), jnp.float32)),
        mesh=mesh,
        scratch_types=[
            pltpu.VMEM_SHARED((Np, d), jnp.float32),
            pltpu.VMEM((3, CH), jnp.int32),
            pltpu.VMEM((3, CH), jnp.int32),
            pltpu.VMEM((CH,), jnp.int32),
            pltpu.VMEM((CH,), jnp.int32),
            pltpu.VMEM((CH,), jnp.float32),
            pltpu.VMEM((CH,), jnp.float32),
            pltpu.VMEM((CH, d), jnp.float32),
            pltpu.VMEM((CH, d), jnp.float32),
            pltpu.VMEM((N,), jnp.float32),
            pltpu.SemaphoreType.DMA,
            pltpu.SemaphoreType.DMA,
            pltpu.SemaphoreType.DMA,
        ],
        compiler_params=pltpu.CompilerParams(needs_layout_passes=False,
                                             use_tc_tiling_on_sc=False),
    )
    return f(h2, epk)


def kernel(node_features, edge_index, edge_features, global_features,
           W1a, b1a, W2a, b2a, W1b, b1b, W2b, b2b, W1c, b1c, W2c, b2c):
    x = node_features[0]
    N, d = x.shape
    hdim = W1a.shape[0]
    src = edge_index[0, 0]
    dst = edge_index[0, 1]
    E = src.shape[0]
    NCH = 2 * E // (NW * CH)

    grid = N // BN
    row_spec = pl.BlockSpec((BN, d), lambda i: (i, 0))

    h2, g = pl.pallas_call(
        _mlp_kernel,
        grid=(grid,),
        in_specs=[
            row_spec,
            _full_spec((d, hdim)), _full_spec((1, hdim)),
            _full_spec((hdim, d)), _full_spec((1, d)),
            _full_spec((1, d)),
            _full_spec((d, hdim)), _full_spec((1, hdim)),
            _full_spec((hdim, d)), _full_spec((1, d)),
        ],
        out_specs=[row_spec, _full_spec((1, d))],
        out_shape=[jax.ShapeDtypeStruct((N, d), jnp.float32),
                   jax.ShapeDtypeStruct((1, d), jnp.float32)],
    )(x, W1b.T, b1b[None], W2b.T, b2b[None],
      global_features[0], W1c.T, b1c[None], W2c.T, b2c[None])

    ebits = lax.bitcast_convert_type(edge_features[0], jnp.int32)
    epk = jnp.stack([
        jnp.concatenate([src, dst]).reshape(NW, NCH, CH),
        jnp.concatenate([dst, src]).reshape(NW, NCH, CH),
        jnp.concatenate([ebits, ebits]).reshape(NW, NCH, CH),
    ], axis=2)

    msg2, den32 = _sc_edge(h2, epk, N)

    out = pl.pallas_call(
        _combine_kernel,
        grid=(grid,),
        in_specs=[
            row_spec,
            pl.BlockSpec((NC, BN, d), lambda i: (0, i, 0)),
            pl.BlockSpec((BN, NW), lambda i: (i, 0)),
            _full_spec((1, d)),
            _full_spec((d, hdim)), _full_spec((1, hdim)),
            _full_spec((hdim, d)), _full_spec((1, d)),
        ],
        out_specs=row_spec,
        out_shape=jax.ShapeDtypeStruct((N, d), jnp.float32),
    )(x, msg2, den32.T, g,
      W1a.T, b1a[None], W2a.T, b2a[None])

    return out[None]
